# Initial kernel scaffold; baseline (speedup 1.0000x reference)
import jax
import jax.numpy as jnp
from jax.experimental import pallas as pl


def kernel(x, batch, pairs_indices, pairs_labels, W1, b1, Wg, att_src, att_dst, bg, W2, b2, W3, b3, W4, b4):
    raise NotImplementedError("write your pallas kernel here")


# speedup vs baseline: 2.7797x; 2.7797x over previous
import functools

import jax
import jax.numpy as jnp
from jax import lax
from jax.experimental import pallas as pl
from jax.experimental.pallas import tpu as pltpu
from jax.experimental.pallas import tpu_sc as plsc

_RB = 128
_CB = 128
_K = 16


def _dot_t(a, b):
    return lax.dot_general(a, b, (((1,), (1,)), ((), ())),
                           preferred_element_type=jnp.float32)


def _pre_body(x_ref, w1_ref, b1_ref, wg_ref, asrc_c_ref, adst_c_ref, asrc_r_ref,
              h_ref, xw_ref, s_pr_ref, t_pr_ref, s_row_ref):
    xb = x_ref[...]
    h = _dot_t(xb, w1_ref[...]) + b1_ref[...]
    xw = _dot_t(h, wg_ref[...])
    h_ref[...] = h
    xw_ref[...] = xw
    s_pr_ref[...] = jnp.dot(xw, asrc_c_ref[...], preferred_element_type=jnp.float32)
    t_pr_ref[...] = jnp.dot(xw, adst_c_ref[...], preferred_element_type=jnp.float32)
    s_row_ref[...] = _dot_t(asrc_r_ref[...], xw).reshape(1, 1, -1)


def _knn_body(sinfo_ref, batch_r_ref, h_r_ref, h_ref, batch2d_ref, nbr_ref):
    b = pl.program_id(0)
    cb0 = sinfo_ref[b, 0]
    ncb = sinfo_ref[b, 1]
    rows = b * _RB + lax.broadcasted_iota(jnp.int32, (_RB, 1), 0)
    batch_r = batch_r_ref[0]
    h_r = h_r_ref[...]
    sq_r = jnp.sum(h_r * h_r, axis=1, keepdims=True)
    iota_cat = lax.broadcasted_iota(jnp.int32, (_RB, _K + _CB), 1)

    def body(j, carry):
        best_d, best_i = carry
        hc = h_ref[pl.ds(j * _CB, _CB), :]
        sq_c = jnp.sum(hc * hc, axis=1)
        d = sq_r + sq_c[None, :] - 2.0 * _dot_t(h_r, hc)
        batch_c = batch2d_ref[j]
        cols = j * _CB + lax.broadcasted_iota(jnp.int32, (1, _CB), 1)
        valid = (batch_r == batch_c[None, :]) & (rows != cols)
        d = jnp.where(valid, d, jnp.inf)
        cand_d = jnp.concatenate([best_d, d], axis=1)
        cand_i = jnp.concatenate([best_i, jnp.broadcast_to(cols, (_RB, _CB))],
                                 axis=1)
        nd, ni = [], []
        for _ in range(_K):
            m = jnp.min(cand_d, axis=1, keepdims=True)
            pos = jnp.min(jnp.where(cand_d == m, iota_cat, jnp.int32(1 << 30)),
                          axis=1, keepdims=True)
            onehot = iota_cat == pos
            sel = jnp.sum(jnp.where(onehot, cand_i, 0), axis=1, keepdims=True)
            nd.append(m)
            ni.append(sel)
            cand_d = jnp.where(onehot, jnp.inf, cand_d)
        return jnp.concatenate(nd, axis=1), jnp.concatenate(ni, axis=1)

    init = (jnp.full((_RB, _K), jnp.inf, jnp.float32),
            jnp.zeros((_RB, _K), jnp.int32))
    _, best_i = lax.fori_loop(cb0, cb0 + ncb, body, init)
    nbr_ref[...] = best_i


def _gat_body(sinfo_ref, nbr_ref, s_pr_ref, t_pr_ref, s_row_ref, xw_ref,
              bg_ref, w2_ref, b2_ref, w3_ref, b3_ref, w4_ref, b4_ref, out_ref):
    b = pl.program_id(0)
    cb0 = sinfo_ref[b, 0]
    ncb = sinfo_ref[b, 1]
    rows = b * _RB + lax.broadcasted_iota(jnp.int32, (_RB, 1), 0)
    nbr = nbr_ref[...]
    s_b = s_pr_ref[...]
    t_b = t_pr_ref[...]

    def leaky(v):
        return jnp.where(v > 0, v, 0.2 * v)

    def g1(j, s_nbr):
        cols = j * _CB + lax.broadcasted_iota(jnp.int32, (1, _CB), 1)
        s_c = s_row_ref[j, 0]
        acc = []
        for t in range(_K):
            match = nbr[:, t:t + 1] == cols
            acc.append(jnp.sum(jnp.where(match, s_c, 0.0), axis=1,
                               keepdims=True))
        return s_nbr + jnp.concatenate(acc, axis=1)

    s_nbr = lax.fori_loop(cb0, cb0 + ncb, g1, jnp.zeros((_RB, _K), jnp.float32))

    e = leaky(s_nbr + t_b)
    e_self = leaky(s_b + t_b)
    m = jnp.maximum(jnp.max(e, axis=1, keepdims=True), e_self)
    ee = jnp.exp(e - m)
    ee_self = jnp.exp(e_self - m)
    denom = jnp.sum(ee, axis=1, keepdims=True) + ee_self + 1e-16
    alpha = ee / denom
    alpha_self = ee_self / denom

    def g2(j, acc):
        cols = j * _CB + lax.broadcasted_iota(jnp.int32, (1, _CB), 1)
        a = jnp.where(rows == cols, alpha_self, 0.0)
        for t in range(_K):
            a = a + jnp.where(nbr[:, t:t + 1] == cols, alpha[:, t:t + 1], 0.0)
        xwc = xw_ref[pl.ds(j * _CB, _CB), :]
        return acc + lax.dot_general(a, xwc, (((1,), (0,)), ((), ())),
                                     preferred_element_type=jnp.float32)

    g = lax.fori_loop(cb0, cb0 + ncb, g2, jnp.zeros((_RB, xw_ref.shape[1]),
                                                    jnp.float32))
    g = g + bg_ref[...]
    h2 = jnp.maximum(_dot_t(g, w2_ref[...]) + b2_ref[...], 0.0)
    h3 = jnp.maximum(_dot_t(h2, w3_ref[...]) + b3_ref[...], 0.0)
    out_ref[...] = _dot_t(h3, w4_ref[...]) + b4_ref[...]


def _pair_gather_sc(table, idx):
    nfo = plsc.get_sparse_core_info()
    nc, ns = nfo.num_cores, nfo.num_subcores
    nw = nc * ns
    b_total, d = idx.shape[0], table.shape[1]
    bpw = b_total // nw
    nchunk = bpw // 128
    mesh = plsc.VectorSubcoreMesh(core_axis_name="c", subcore_axis_name="s")

    @functools.partial(
        pl.kernel, mesh=mesh,
        out_type=jax.ShapeDtypeStruct((b_total, d), jnp.float32),
        scratch_types=[
            pltpu.VMEM((nchunk, 128), jnp.int32),
            pltpu.VMEM((bpw, d), jnp.float32),
            pltpu.SemaphoreType.DMA,
        ],
    )
    def k(table_hbm, idx_hbm, out_hbm, idx_v, rows_v, sem):
        wid = lax.axis_index("s") * nc + lax.axis_index("c")
        base = wid * bpw
        pltpu.sync_copy(idx_hbm.at[wid], idx_v)
        for c in range(nchunk):
            pltpu.async_copy(table_hbm.at[idx_v.at[c]],
                             rows_v.at[pl.ds(c * 128, 128)], sem).wait()
        pltpu.sync_copy(rows_v, out_hbm.at[pl.ds(base, bpw)])

    return k(table, idx.reshape(nw, nchunk, 128))


def kernel(x, batch, pairs_indices, pairs_labels, W1, b1, Wg, att_src, att_dst,
           bg, W2, b2, W3, b3, W4, b4):
    n, d_in = x.shape
    hid = W1.shape[0]
    nb = (n + _RB - 1) // _RB
    npad = nb * _RB

    xp = jnp.pad(x, ((0, npad - n), (0, 0)))
    batch_p = jnp.pad(batch.astype(jnp.int32), (0, npad - n),
                      constant_values=-1)

    idx_lo = jnp.minimum(jnp.arange(nb, dtype=jnp.int32) * _RB, n - 1)
    idx_hi = jnp.minimum(idx_lo + _RB - 1, n - 1)
    cs = jnp.searchsorted(batch, batch[idx_lo], side="left").astype(jnp.int32)
    ce = jnp.searchsorted(batch, batch[idx_hi], side="right").astype(jnp.int32)
    cb0 = cs // _CB
    ncb = (ce + _CB - 1) // _CB - cb0
    sinfo = jnp.stack([cb0, ncb], axis=1)

    f32 = jnp.float32
    grid = (nb,)
    row_spec = lambda lastdim: pl.BlockSpec((_RB, lastdim), lambda b_: (b_, 0))
    full = pl.BlockSpec((None, None))

    def whole(shape_arr):
        return pl.BlockSpec(shape_arr.shape, lambda b_: (0,) * shape_arr.ndim)

    h, xw, s_pr, t_pr, s_row = pl.pallas_call(
        _pre_body,
        grid=grid,
        in_specs=[row_spec(d_in), whole(W1), whole(b1.reshape(1, hid)),
                  whole(Wg), whole(att_src.reshape(hid, 1)),
                  whole(att_dst.reshape(hid, 1)),
                  whole(att_src.reshape(1, hid))],
        out_specs=[row_spec(hid), row_spec(hid), row_spec(1), row_spec(1),
                   pl.BlockSpec((1, 1, _RB), lambda b_: (b_, 0, 0))],
        out_shape=[jax.ShapeDtypeStruct((npad, hid), f32),
                   jax.ShapeDtypeStruct((npad, hid), f32),
                   jax.ShapeDtypeStruct((npad, 1), f32),
                   jax.ShapeDtypeStruct((npad, 1), f32),
                   jax.ShapeDtypeStruct((nb, 1, _RB), f32)],
    )(xp, W1, b1.reshape(1, hid), Wg, att_src.reshape(hid, 1),
      att_dst.reshape(hid, 1), att_src.reshape(1, hid))

    nbr = pl.pallas_call(
        _knn_body,
        grid=grid,
        in_specs=[pl.BlockSpec(memory_space=pltpu.SMEM),
                  pl.BlockSpec((1, _RB, 1), lambda b_: (b_, 0, 0)),
                  row_spec(hid), whole(h),
                  whole(batch_p.reshape(nb, _RB))],
        out_specs=pl.BlockSpec((_RB, _K), lambda b_: (b_, 0)),
        out_shape=jax.ShapeDtypeStruct((npad, _K), jnp.int32),
    )(sinfo, batch_p.reshape(nb, _RB, 1), h, h, batch_p.reshape(nb, _RB))

    hfin = pl.pallas_call(
        _gat_body,
        grid=grid,
        in_specs=[pl.BlockSpec(memory_space=pltpu.SMEM),
                  pl.BlockSpec((_RB, _K), lambda b_: (b_, 0)),
                  row_spec(1), row_spec(1),
                  whole(s_row), whole(xw),
                  whole(bg.reshape(1, hid)),
                  whole(W2), whole(b2.reshape(1, b2.shape[0])),
                  whole(W3), whole(b3.reshape(1, b3.shape[0])),
                  whole(W4), whole(b4.reshape(1, b4.shape[0]))],
        out_specs=row_spec(W4.shape[0]),
        out_shape=jax.ShapeDtypeStruct((npad, W4.shape[0]), f32),
    )(sinfo, nbr, s_pr, t_pr, s_row, xw, bg.reshape(1, hid), W2,
      b2.reshape(1, b2.shape[0]), W3, b3.reshape(1, b3.shape[0]), W4,
      b4.reshape(1, b4.shape[0]))

    npairs = pairs_indices.shape[0]
    idx_flat = jnp.concatenate([pairs_indices[:, 0], pairs_indices[:, 1]]
                               ).astype(jnp.int32)
    pairs = _pair_gather_sc(hfin, idx_flat)
    pair_embeddings = pairs.reshape(2, npairs, W4.shape[0])
    return pair_embeddings, pairs_labels


# speedup vs baseline: 7.4746x; 2.6890x over previous
import functools

import jax
import jax.numpy as jnp
from jax import lax
from jax.experimental import pallas as pl
from jax.experimental.pallas import tpu as pltpu
from jax.experimental.pallas import tpu_sc as plsc

_RB = 128
_CB = 128
_K = 16


def _dot_t(a, b):
    return lax.dot_general(a, b, (((1,), (1,)), ((), ())),
                           preferred_element_type=jnp.float32)


def _pre_body(x_ref, w1_ref, b1_ref, wg_ref, h_ref, xw_ref):
    xb = x_ref[...]
    h = _dot_t(xb, w1_ref[...]) + b1_ref[...]
    h_ref[...] = h
    xw_ref[...] = _dot_t(h, wg_ref[...])


def _knn_body(sinfo_ref, batch_r_ref, h_r_ref, h_ref, batch2d_ref, nbr_ref):
    b = pl.program_id(0)
    cb0 = sinfo_ref[b, 0]
    ncb = sinfo_ref[b, 1]
    rows = b * _RB + lax.broadcasted_iota(jnp.int32, (_RB, 1), 0)
    batch_r = batch_r_ref[0]
    h_r = h_r_ref[...]
    sq_r = jnp.sum(h_r * h_r, axis=1, keepdims=True)

    def body(j, carry):
        best_d, best_i = carry
        hc = h_ref[pl.ds(j * _CB, _CB), :]
        sq_c = jnp.sum(hc * hc, axis=1)
        d = sq_r + sq_c - 2.0 * _dot_t(h_r, hc)
        batch_c = batch2d_ref[j]
        cols = j * _CB + lax.broadcasted_iota(jnp.int32, (1, _CB), 1)
        valid = (batch_r == batch_c[None, :]) & (rows != cols)
        d = jnp.where(valid, d, jnp.inf)
        cand_d = jnp.concatenate([best_d, d], axis=1)
        cand_i = jnp.concatenate([best_i, jnp.broadcast_to(cols, (_RB, _CB))],
                                 axis=1)
        nd, ni = [], []
        for _ in range(_K):
            m = jnp.min(cand_d, axis=1, keepdims=True)
            onehot = cand_d == m
            sel = jnp.min(jnp.where(onehot, cand_i, jnp.int32(1 << 30)),
                          axis=1, keepdims=True)
            nd.append(m)
            ni.append(sel)
            cand_d = jnp.where(onehot, jnp.inf, cand_d)
        return jnp.concatenate(nd, axis=1), jnp.concatenate(ni, axis=1)

    init = (jnp.full((_RB, _K), jnp.inf, jnp.float32),
            jnp.zeros((_RB, _K), jnp.int32))
    _, best_i = lax.fori_loop(cb0, cb0 + ncb, body, init)
    nbr_ref[...] = best_i


def _gat_body(g3_ref, asrc_ref, adst_ref, bg_ref, w2_ref, b2_ref, w3_ref,
              b3_ref, w4_ref, b4_ref, out_ref):
    hid = w2_ref.shape[1]
    xw_self = g3_ref[:, _K, :]
    t_b = jnp.dot(xw_self, adst_ref[...], preferred_element_type=jnp.float32)
    s_nbr = jnp.concatenate(
        [jnp.dot(g3_ref[:, t, :], asrc_ref[...],
                 preferred_element_type=jnp.float32)
         for t in range(_K + 1)], axis=1)
    e = s_nbr + t_b
    e = jnp.where(e > 0, e, 0.2 * e)
    m = jnp.max(e, axis=1, keepdims=True)
    ee = jnp.exp(e - m)
    denom = jnp.sum(ee, axis=1, keepdims=True) + 1e-16
    alpha = ee / denom
    acc = jnp.zeros((_RB, hid), jnp.float32)
    for t in range(_K + 1):
        acc = acc + alpha[:, t:t + 1] * g3_ref[:, t, :]
    g = acc + bg_ref[...]
    h2 = jnp.maximum(_dot_t(g, w2_ref[...]) + b2_ref[...], 0.0)
    h3 = jnp.maximum(_dot_t(h2, w3_ref[...]) + b3_ref[...], 0.0)
    out_ref[...] = _dot_t(h3, w4_ref[...]) + b4_ref[...]


def _sc_gather(table, idx):
    nfo = plsc.get_sparse_core_info()
    nc, ns = nfo.num_cores, nfo.num_subcores
    nw = nc * ns
    m_total, d = idx.shape[0], table.shape[1]
    bpw = m_total // nw
    nchunk = bpw // 128
    mesh = plsc.VectorSubcoreMesh(core_axis_name="c", subcore_axis_name="s")

    @functools.partial(
        pl.kernel, mesh=mesh,
        out_type=jax.ShapeDtypeStruct((m_total, d), jnp.float32),
        scratch_types=[
            pltpu.VMEM((nchunk, 128), jnp.int32),
            pltpu.VMEM((128, d), jnp.float32),
            pltpu.VMEM((128, d), jnp.float32),
            pltpu.SemaphoreType.DMA,
            pltpu.SemaphoreType.DMA,
            pltpu.SemaphoreType.DMA,
            pltpu.SemaphoreType.DMA,
        ],
    )
    def k(table_hbm, idx_hbm, out_hbm, idx_v, buf0, buf1, g0, g1, s0, s1):
        wid = lax.axis_index("s") * nc + lax.axis_index("c")
        pltpu.sync_copy(idx_hbm.at[wid], idx_v)
        bufs, gsems, ssems = (buf0, buf1), (g0, g1), (s0, s1)
        gd = [None, None]
        sd = [None, None]
        gd[0] = pltpu.async_copy(table_hbm.at[idx_v.at[0]], bufs[0], gsems[0])
        for c in range(nchunk):
            cur = c & 1
            gd[cur].wait()
            if c + 1 < nchunk:
                nxt = (c + 1) & 1
                if sd[nxt] is not None:
                    sd[nxt].wait()
                gd[nxt] = pltpu.async_copy(table_hbm.at[idx_v.at[c + 1]],
                                           bufs[nxt], gsems[nxt])
            sd[cur] = pltpu.async_copy(
                bufs[cur], out_hbm.at[pl.ds((wid * nchunk + c) * 128, 128)],
                ssems[cur])
        for bb in range(2):
            if sd[bb] is not None:
                sd[bb].wait()

    return k(table, idx.reshape(nw, nchunk, 128))


def kernel(x, batch, pairs_indices, pairs_labels, W1, b1, Wg, att_src, att_dst,
           bg, W2, b2, W3, b3, W4, b4):
    n, d_in = x.shape
    hid = W1.shape[0]
    nb = (n + _RB - 1) // _RB
    npad = nb * _RB

    xp = jnp.pad(x, ((0, npad - n), (0, 0)))
    batch_p = jnp.pad(batch.astype(jnp.int32), (0, npad - n),
                      constant_values=-1)

    idx_lo = jnp.minimum(jnp.arange(nb, dtype=jnp.int32) * _RB, n - 1)
    idx_hi = jnp.minimum(idx_lo + _RB - 1, n - 1)
    cs = jnp.searchsorted(batch, batch[idx_lo], side="left").astype(jnp.int32)
    ce = jnp.searchsorted(batch, batch[idx_hi], side="right").astype(jnp.int32)
    cb0 = cs // _CB
    ncb = (ce + _CB - 1) // _CB - cb0
    sinfo = jnp.stack([cb0, ncb], axis=1)

    f32 = jnp.float32
    grid = (nb,)
    row_spec = lambda lastdim: pl.BlockSpec((_RB, lastdim), lambda b_: (b_, 0))

    def whole(shape_arr):
        return pl.BlockSpec(shape_arr.shape, lambda b_: (0,) * shape_arr.ndim)

    h, xw = pl.pallas_call(
        _pre_body,
        grid=grid,
        in_specs=[row_spec(d_in), whole(W1), whole(b1.reshape(1, hid)),
                  whole(Wg)],
        out_specs=[row_spec(hid), row_spec(hid)],
        out_shape=[jax.ShapeDtypeStruct((npad, hid), f32),
                   jax.ShapeDtypeStruct((npad, hid), f32)],
    )(xp, W1, b1.reshape(1, hid), Wg)

    nbr = pl.pallas_call(
        _knn_body,
        grid=grid,
        in_specs=[pl.BlockSpec(memory_space=pltpu.SMEM),
                  pl.BlockSpec((1, _RB, 1), lambda b_: (b_, 0, 0)),
                  row_spec(hid), whole(h),
                  whole(batch_p.reshape(nb, _RB))],
        out_specs=pl.BlockSpec((_RB, _K), lambda b_: (b_, 0)),
        out_shape=jax.ShapeDtypeStruct((npad, _K), jnp.int32),
    )(sinfo, batch_p.reshape(nb, _RB, 1), h, h, batch_p.reshape(nb, _RB))

    idxg = jnp.concatenate(
        [nbr, jnp.arange(npad, dtype=jnp.int32)[:, None]], axis=1).reshape(-1)
    m_nodes = idxg.shape[0]
    m_pad = -m_nodes % (32 * 128)
    idxg = jnp.pad(idxg, (0, m_pad))
    gflat = _sc_gather(xw, idxg)
    g3 = gflat[:m_nodes].reshape(npad, _K + 1, hid)

    hfin = pl.pallas_call(
        _gat_body,
        grid=grid,
        in_specs=[pl.BlockSpec((_RB, _K + 1, hid), lambda b_: (b_, 0, 0)),
                  whole(att_src.reshape(hid, 1)), whole(att_dst.reshape(hid, 1)),
                  whole(bg.reshape(1, hid)),
                  whole(W2), whole(b2.reshape(1, b2.shape[0])),
                  whole(W3), whole(b3.reshape(1, b3.shape[0])),
                  whole(W4), whole(b4.reshape(1, b4.shape[0]))],
        out_specs=row_spec(W4.shape[0]),
        out_shape=jax.ShapeDtypeStruct((npad, W4.shape[0]), f32),
    )(g3, att_src.reshape(hid, 1), att_dst.reshape(hid, 1), bg.reshape(1, hid),
      W2, b2.reshape(1, b2.shape[0]), W3, b3.reshape(1, b3.shape[0]), W4,
      b4.reshape(1, b4.shape[0]))

    npairs = pairs_indices.shape[0]
    idx_flat = jnp.concatenate([pairs_indices[:, 0], pairs_indices[:, 1]]
                               ).astype(jnp.int32)
    pairs = _sc_gather(hfin, idx_flat)
    pair_embeddings = pairs.reshape(2, npairs, W4.shape[0])
    return pair_embeddings, pairs_labels
